# Initial kernel scaffold; baseline (speedup 1.0000x reference)
import jax
import jax.numpy as jnp
from jax.experimental import pallas as pl


def kernel(hidden_states, gate_W, W1, b1, W2, b2, ln_gamma, ln_beta):
    raise NotImplementedError("write your pallas kernel here")


# speedup vs baseline: 1.9615x; 1.9615x over previous
import functools

import jax
import jax.numpy as jnp
from jax import lax
from jax.experimental import pallas as pl
from jax.experimental.pallas import tpu as pltpu
from jax.experimental.pallas import tpu_sc as plsc

E = 8
KTOP = 2
D = 1024
I = 2048
L = 1024
T = 8192

TM = 512
PBUF = T * KTOP + E * TM
NT = PBUF // TM
NT_PAD = 64
BJ = 1024
NJ = I // BJ

_CHUNK = 256


def _router_body(x_ref, gwt_ref, logits_ref):
    logits_ref[...] = lax.dot_general(
        x_ref[...], gwt_ref[...], (((1,), (0,)), ((), ())),
        preferred_element_type=jnp.float32,
        precision=lax.Precision.DEFAULT)


def _meta_body(logits_ref, pos0_ref, pos1_ref, w0_ref, w1_ref, texp_ref,
               cum_ref):
    lg = logits_ref[...]
    m = jnp.max(lg, axis=1, keepdims=True)
    p = jnp.exp(lg - m)
    probs = p / jnp.sum(p, axis=1, keepdims=True)
    colid = lax.broadcasted_iota(jnp.int32, (T, E), 1)
    m1 = jnp.max(probs, axis=1, keepdims=True)
    e0 = jnp.min(jnp.where(probs == m1, colid, E), axis=1, keepdims=True)
    probs2 = jnp.where(colid == e0, -1.0, probs)
    m2 = jnp.max(probs2, axis=1, keepdims=True)
    e1 = jnp.min(jnp.where(probs2 == m2, colid, E), axis=1, keepdims=True)
    wsum = m1 + m2
    w0_ref[...] = jnp.broadcast_to(m1 / wsum, (T, 16))
    w1_ref[...] = jnp.broadcast_to(m2 / wsum, (T, 16))

    oh0 = (colid == e0).astype(jnp.float32)
    oh1 = (colid == e1).astype(jnp.float32)
    a = oh0 + oh1

    ri = lax.broadcasted_iota(jnp.int32, (_CHUNK, _CHUNK), 0)
    ci = lax.broadcasted_iota(jnp.int32, (_CHUNK, _CHUNK), 1)
    tril_ex = (ri > ci).astype(jnp.float32)

    cum_ref[...] = a

    def body(b, carry):
        ab = cum_ref[pl.ds(b * _CHUNK, _CHUNK), :]
        within = lax.dot_general(tril_ex, ab, (((1,), (0,)), ((), ())),
                                 preferred_element_type=jnp.float32)
        cum_ref[pl.ds(b * _CHUNK, _CHUNK), :] = within + carry
        return carry + jnp.sum(ab, axis=0, keepdims=True)

    counts = lax.fori_loop(0, T // _CHUNK, body, jnp.zeros((1, E), jnp.float32))
    cnt_i = counts.astype(jnp.int32)
    aligned = ((cnt_i + (TM - 1)) // TM * TM).astype(jnp.float32)
    r8 = lax.broadcasted_iota(jnp.int32, (E, E), 0)
    c8 = lax.broadcasted_iota(jnp.int32, (E, E), 1)
    lt8 = (r8 < c8).astype(jnp.float32)
    ps = lax.dot_general(aligned, lt8, (((1,), (0,)), ((), ())),
                         preferred_element_type=jnp.float32)

    cum = cum_ref[...]
    base = jnp.broadcast_to(ps, (T, E))
    pos0 = jnp.sum((cum + base) * oh0, axis=1, keepdims=True)
    pos1 = jnp.sum((cum + base) * oh1, axis=1, keepdims=True)
    pos0_ref[...] = pos0.astype(jnp.int32)
    pos1_ref[...] = pos1.astype(jnp.int32)

    ends = ps + aligned
    tb = (lax.broadcasted_iota(jnp.int32, (NT_PAD, E), 0) * TM).astype(
        jnp.float32)
    texp_ref[...] = jnp.sum((tb >= jnp.broadcast_to(ends, (NT_PAD, E)))
                            .astype(jnp.int32), axis=1, keepdims=True)


def _ffn_body(texp, xs_ref, w1_ref, b1_ref, w2_ref, b2_ref, g_ref, bt_ref,
              out_ref):
    i = pl.program_id(0)
    j = pl.program_id(1)
    e = texp[i]

    @pl.when(e < E)
    def _():
        x = xs_ref[...]
        h = lax.dot_general(x, w1_ref[0], (((1,), (0,)), ((), ())),
                            preferred_element_type=jnp.float32)
        h = jnp.maximum(h + b1_ref[0], 0.0)
        contrib = lax.dot_general(h, w2_ref[0], (((1,), (0,)), ((), ())),
                                  preferred_element_type=jnp.float32)

        @pl.when(j == 0)
        def _():
            out_ref[...] = contrib

        @pl.when(j > 0)
        def _():
            out_ref[...] += contrib

        @pl.when(j == NJ - 1)
        def _():
            o = out_ref[...] + b2_ref[0]
            mu = jnp.mean(o, axis=1, keepdims=True)
            oc = o - mu
            var = jnp.mean(oc * oc, axis=1, keepdims=True)
            o = oc * lax.rsqrt(var + 1e-5)
            out_ref[...] = o * g_ref[0] + bt_ref[0]


_NC, _NS = 2, 16
_NW = _NC * _NS
_TPW = T // _NW
_SCH = 64
_CCH = 32


def _sc_scatter_body(x_hbm, pos0_hbm, pos1_hbm, xs_hbm, xv, i0v, i1v,
                     sem0, sem1):
    wid = lax.axis_index("s") * _NC + lax.axis_index("c")
    base = wid * _TPW

    def chunk(ci, _):
        tb = base + ci * _SCH
        pltpu.sync_copy(x_hbm.at[pl.ds(tb, _SCH)], xv)
        pltpu.sync_copy(pos0_hbm.at[pl.ds(tb, _SCH)], i0v)
        pltpu.sync_copy(pos1_hbm.at[pl.ds(tb, _SCH)], i1v)
        cp0 = pltpu.async_copy(xv, xs_hbm.at[i0v], sem0)
        cp1 = pltpu.async_copy(xv, xs_hbm.at[i1v], sem1)
        cp0.wait()
        cp1.wait()
        return 0

    lax.fori_loop(0, _TPW // _SCH, chunk, 0)


_sc_scatter = pl.kernel(
    _sc_scatter_body,
    out_type=jax.ShapeDtypeStruct((PBUF, D), jnp.float32),
    mesh=plsc.VectorSubcoreMesh(core_axis_name="c", subcore_axis_name="s"),
    scratch_types=[
        pltpu.VMEM((_SCH, D), jnp.float32),
        pltpu.VMEM((_SCH,), jnp.int32),
        pltpu.VMEM((_SCH,), jnp.int32),
        pltpu.SemaphoreType.DMA,
        pltpu.SemaphoreType.DMA,
    ],
)


def _sc_combine_body(o_hbm, pos0_hbm, pos1_hbm, w0_hbm, w1_hbm, out_hbm,
                     i0v, i1v, w0v, w1v, b0, b1, ov, sem0, sem1):
    wid = lax.axis_index("s") * _NC + lax.axis_index("c")
    base = wid * _TPW

    def chunk(ci, _):
        tb = base + ci * _CCH
        pltpu.sync_copy(pos0_hbm.at[pl.ds(tb, _CCH)], i0v)
        pltpu.sync_copy(pos1_hbm.at[pl.ds(tb, _CCH)], i1v)
        pltpu.sync_copy(w0_hbm.at[pl.ds(tb, _CCH)], w0v)
        pltpu.sync_copy(w1_hbm.at[pl.ds(tb, _CCH)], w1v)
        cp0 = pltpu.async_copy(o_hbm.at[i0v], b0, sem0)
        cp1 = pltpu.async_copy(o_hbm.at[i1v], b1, sem1)
        cp0.wait()
        cp1.wait()

        def row(r, _):
            w0r = w0v[r, pl.ds(0, 16)]
            w1r = w1v[r, pl.ds(0, 16)]
            for c in range(L // 16):
                ov[r, pl.ds(c * 16, 16)] = (
                    w0r * b0[r, pl.ds(c * 16, 16)]
                    + w1r * b1[r, pl.ds(c * 16, 16)])
            return 0

        lax.fori_loop(0, _CCH, row, 0)
        pltpu.sync_copy(ov, out_hbm.at[pl.ds(tb, _CCH)])
        return 0

    lax.fori_loop(0, _TPW // _CCH, chunk, 0)


_sc_combine = pl.kernel(
    _sc_combine_body,
    out_type=jax.ShapeDtypeStruct((T, L), jnp.float32),
    mesh=plsc.VectorSubcoreMesh(core_axis_name="c", subcore_axis_name="s"),
    scratch_types=[
        pltpu.VMEM((_CCH,), jnp.int32),
        pltpu.VMEM((_CCH,), jnp.int32),
        pltpu.VMEM((_CCH, 16), jnp.float32),
        pltpu.VMEM((_CCH, 16), jnp.float32),
        pltpu.VMEM((_CCH, L), jnp.float32),
        pltpu.VMEM((_CCH, L), jnp.float32),
        pltpu.VMEM((_CCH, L), jnp.float32),
        pltpu.SemaphoreType.DMA,
        pltpu.SemaphoreType.DMA,
    ],
)


def _clampe(e):
    return jnp.minimum(e, E - 1)


def _ffn_call(xs, W1, b1, W2, b2, ln_gamma, ln_beta, texp):
    grid_spec = pltpu.PrefetchScalarGridSpec(
        num_scalar_prefetch=1,
        grid=(NT, NJ),
        in_specs=[
            pl.BlockSpec((TM, D), lambda i, j, t: (i, 0)),
            pl.BlockSpec((1, D, BJ), lambda i, j, t: (_clampe(t[i]), 0, j)),
            pl.BlockSpec((1, 1, BJ), lambda i, j, t: (_clampe(t[i]), 0, j)),
            pl.BlockSpec((1, BJ, L), lambda i, j, t: (_clampe(t[i]), j, 0)),
            pl.BlockSpec((1, 1, L), lambda i, j, t: (_clampe(t[i]), 0, 0)),
            pl.BlockSpec((1, 1, L), lambda i, j, t: (_clampe(t[i]), 0, 0)),
            pl.BlockSpec((1, 1, L), lambda i, j, t: (_clampe(t[i]), 0, 0)),
        ],
        out_specs=pl.BlockSpec((TM, L), lambda i, j, t: (i, 0)),
    )
    return pl.pallas_call(
        _ffn_body,
        grid_spec=grid_spec,
        out_shape=jax.ShapeDtypeStruct((PBUF, L), jnp.float32),
    )(texp, xs, W1, b1.reshape(E, 1, I), W2, b2.reshape(E, 1, L),
      ln_gamma.reshape(E, 1, L), ln_beta.reshape(E, 1, L))


def kernel(hidden_states, gate_W, W1, b1, W2, b2, ln_gamma, ln_beta):
    x = hidden_states.reshape(T, D)

    logits = pl.pallas_call(
        _router_body,
        grid=(T // 512,),
        in_specs=[
            pl.BlockSpec((512, D), lambda i: (i, 0)),
            pl.BlockSpec((D, E), lambda i: (0, 0)),
        ],
        out_specs=pl.BlockSpec((512, E), lambda i: (i, 0)),
        out_shape=jax.ShapeDtypeStruct((T, E), jnp.float32),
    )(x, gate_W.T)

    pos0, pos1, w0, w1, texp = pl.pallas_call(
        _meta_body,
        out_shape=[
            jax.ShapeDtypeStruct((T, 1), jnp.int32),
            jax.ShapeDtypeStruct((T, 1), jnp.int32),
            jax.ShapeDtypeStruct((T, 16), jnp.float32),
            jax.ShapeDtypeStruct((T, 16), jnp.float32),
            jax.ShapeDtypeStruct((NT_PAD, 1), jnp.int32),
        ],
        scratch_shapes=[pltpu.VMEM((T, E), jnp.float32)],
    )(logits)

    pos0f = pos0.reshape(T)
    pos1f = pos1.reshape(T)

    xs = _sc_scatter(x, pos0f, pos1f)

    o_sorted = _ffn_call(xs, W1, b1, W2, b2, ln_gamma, ln_beta,
                         texp.reshape(NT_PAD)[:NT])

    final = _sc_combine(o_sorted, pos0f, pos1f, w0, w1)

    return (final.reshape(hidden_states.shape[0], hidden_states.shape[1], L),
            logits)


# speedup vs baseline: 2.1344x; 1.0882x over previous
import functools

import jax
import jax.numpy as jnp
from jax import lax
from jax.experimental import pallas as pl
from jax.experimental.pallas import tpu as pltpu
from jax.experimental.pallas import tpu_sc as plsc

E = 8
KTOP = 2
D = 1024
I = 2048
L = 1024
T = 8192

TM = 512
PBUF = T * KTOP + E * TM
NT = PBUF // TM
NT_PAD = 64
BJ = 2048
NJ = I // BJ

_CHUNK = 256


def _router_body(x_ref, gwt_ref, logits_ref):
    logits_ref[...] = lax.dot_general(
        x_ref[...], gwt_ref[...], (((1,), (0,)), ((), ())),
        preferred_element_type=jnp.float32,
        precision=lax.Precision.DEFAULT)


def _meta_body(logits_ref, pos0_ref, pos1_ref, w0_ref, w1_ref, texp_ref,
               cum_ref):
    lg = logits_ref[...]
    m = jnp.max(lg, axis=1, keepdims=True)
    p = jnp.exp(lg - m)
    probs = p / jnp.sum(p, axis=1, keepdims=True)
    colid = lax.broadcasted_iota(jnp.int32, (T, E), 1)
    m1 = jnp.max(probs, axis=1, keepdims=True)
    e0 = jnp.min(jnp.where(probs == m1, colid, E), axis=1, keepdims=True)
    probs2 = jnp.where(colid == e0, -1.0, probs)
    m2 = jnp.max(probs2, axis=1, keepdims=True)
    e1 = jnp.min(jnp.where(probs2 == m2, colid, E), axis=1, keepdims=True)
    wsum = m1 + m2
    w0_ref[...] = jnp.broadcast_to(m1 / wsum, (T, 16))
    w1_ref[...] = jnp.broadcast_to(m2 / wsum, (T, 16))

    oh0 = (colid == e0).astype(jnp.float32)
    oh1 = (colid == e1).astype(jnp.float32)
    a = oh0 + oh1

    ri = lax.broadcasted_iota(jnp.int32, (_CHUNK, _CHUNK), 0)
    ci = lax.broadcasted_iota(jnp.int32, (_CHUNK, _CHUNK), 1)
    tril_ex = (ri > ci).astype(jnp.float32)

    cum_ref[...] = a

    def body(b, carry):
        ab = cum_ref[pl.ds(b * _CHUNK, _CHUNK), :]
        within = lax.dot_general(tril_ex, ab, (((1,), (0,)), ((), ())),
                                 preferred_element_type=jnp.float32)
        cum_ref[pl.ds(b * _CHUNK, _CHUNK), :] = within + carry
        return carry + jnp.sum(ab, axis=0, keepdims=True)

    counts = lax.fori_loop(0, T // _CHUNK, body, jnp.zeros((1, E), jnp.float32))
    cnt_i = counts.astype(jnp.int32)
    aligned = ((cnt_i + (TM - 1)) // TM * TM).astype(jnp.float32)
    r8 = lax.broadcasted_iota(jnp.int32, (E, E), 0)
    c8 = lax.broadcasted_iota(jnp.int32, (E, E), 1)
    lt8 = (r8 < c8).astype(jnp.float32)
    ps = lax.dot_general(aligned, lt8, (((1,), (0,)), ((), ())),
                         preferred_element_type=jnp.float32)

    cum = cum_ref[...]
    base = jnp.broadcast_to(ps, (T, E))
    pos0 = jnp.sum((cum + base) * oh0, axis=1, keepdims=True)
    pos1 = jnp.sum((cum + base) * oh1, axis=1, keepdims=True)
    pos0_ref[...] = pos0.astype(jnp.int32)
    pos1_ref[...] = pos1.astype(jnp.int32)

    ends = ps + aligned
    tb = (lax.broadcasted_iota(jnp.int32, (NT_PAD, E), 0) * TM).astype(
        jnp.float32)
    texp_ref[...] = jnp.sum((tb >= jnp.broadcast_to(ends, (NT_PAD, E)))
                            .astype(jnp.int32), axis=1, keepdims=True)


def _ffn_body(texp, xs_ref, w1_ref, b1_ref, w2_ref, b2_ref, g_ref, bt_ref,
              out_ref):
    i = pl.program_id(0)
    j = pl.program_id(1)
    e = texp[i]

    @pl.when(e < E)
    def _():
        x = xs_ref[...].astype(jnp.bfloat16)
        h = lax.dot_general(x, w1_ref[0], (((1,), (0,)), ((), ())),
                            preferred_element_type=jnp.float32)
        h = jnp.maximum(h + b1_ref[0], 0.0).astype(jnp.bfloat16)
        contrib = lax.dot_general(h, w2_ref[0], (((1,), (0,)), ((), ())),
                                  preferred_element_type=jnp.float32)

        @pl.when(j == 0)
        def _():
            out_ref[...] = contrib

        @pl.when(j > 0)
        def _():
            out_ref[...] += contrib

        @pl.when(j == NJ - 1)
        def _():
            o = out_ref[...] + b2_ref[0]
            mu = jnp.mean(o, axis=1, keepdims=True)
            oc = o - mu
            var = jnp.mean(oc * oc, axis=1, keepdims=True)
            o = oc * lax.rsqrt(var + 1e-5)
            out_ref[...] = o * g_ref[0] + bt_ref[0]


_NC, _NS = 2, 16
_NW = _NC * _NS
_TPW = T // _NW
_SCH = 64
_CCH = 32


def _sc_scatter_body(x_hbm, pos0_hbm, pos1_hbm, xs_hbm, xv, i0v, i1v,
                     sem0, sem1):
    wid = lax.axis_index("s") * _NC + lax.axis_index("c")
    base = wid * _TPW

    def chunk(ci, _):
        tb = base + ci * _SCH
        pltpu.sync_copy(x_hbm.at[pl.ds(tb, _SCH)], xv)
        pltpu.sync_copy(pos0_hbm.at[pl.ds(tb, _SCH)], i0v)
        pltpu.sync_copy(pos1_hbm.at[pl.ds(tb, _SCH)], i1v)
        cp0 = pltpu.async_copy(xv, xs_hbm.at[i0v], sem0)
        cp1 = pltpu.async_copy(xv, xs_hbm.at[i1v], sem1)
        cp0.wait()
        cp1.wait()
        return 0

    lax.fori_loop(0, _TPW // _SCH, chunk, 0)


_sc_scatter = pl.kernel(
    _sc_scatter_body,
    out_type=jax.ShapeDtypeStruct((PBUF, D), jnp.float32),
    mesh=plsc.VectorSubcoreMesh(core_axis_name="c", subcore_axis_name="s"),
    scratch_types=[
        pltpu.VMEM((_SCH, D), jnp.float32),
        pltpu.VMEM((_SCH,), jnp.int32),
        pltpu.VMEM((_SCH,), jnp.int32),
        pltpu.SemaphoreType.DMA,
        pltpu.SemaphoreType.DMA,
    ],
)


def _sc_combine_body(o_hbm, pos0_hbm, pos1_hbm, w0_hbm, w1_hbm, out_hbm,
                     i0v, i1v, w0v, w1v, b0, b1, ov, sem0, sem1):
    wid = lax.axis_index("s") * _NC + lax.axis_index("c")
    base = wid * _TPW

    def chunk(ci, _):
        tb = base + ci * _CCH
        pltpu.sync_copy(pos0_hbm.at[pl.ds(tb, _CCH)], i0v)
        pltpu.sync_copy(pos1_hbm.at[pl.ds(tb, _CCH)], i1v)
        pltpu.sync_copy(w0_hbm.at[pl.ds(tb, _CCH)], w0v)
        pltpu.sync_copy(w1_hbm.at[pl.ds(tb, _CCH)], w1v)
        cp0 = pltpu.async_copy(o_hbm.at[i0v], b0, sem0)
        cp1 = pltpu.async_copy(o_hbm.at[i1v], b1, sem1)
        cp0.wait()
        cp1.wait()

        def row(r, _):
            w0r = w0v[r, pl.ds(0, 16)]
            w1r = w1v[r, pl.ds(0, 16)]
            for c in range(L // 16):
                ov[r, pl.ds(c * 16, 16)] = (
                    w0r * b0[r, pl.ds(c * 16, 16)]
                    + w1r * b1[r, pl.ds(c * 16, 16)])
            return 0

        lax.fori_loop(0, _CCH, row, 0)
        pltpu.sync_copy(ov, out_hbm.at[pl.ds(tb, _CCH)])
        return 0

    lax.fori_loop(0, _TPW // _CCH, chunk, 0)


_sc_combine = pl.kernel(
    _sc_combine_body,
    out_type=jax.ShapeDtypeStruct((T, L), jnp.float32),
    mesh=plsc.VectorSubcoreMesh(core_axis_name="c", subcore_axis_name="s"),
    scratch_types=[
        pltpu.VMEM((_CCH,), jnp.int32),
        pltpu.VMEM((_CCH,), jnp.int32),
        pltpu.VMEM((_CCH, 16), jnp.float32),
        pltpu.VMEM((_CCH, 16), jnp.float32),
        pltpu.VMEM((_CCH, L), jnp.float32),
        pltpu.VMEM((_CCH, L), jnp.float32),
        pltpu.VMEM((_CCH, L), jnp.float32),
        pltpu.SemaphoreType.DMA,
        pltpu.SemaphoreType.DMA,
    ],
)


def _clampe(e):
    return jnp.minimum(e, E - 1)


def _ffn_call(xs, W1, b1, W2, b2, ln_gamma, ln_beta, texp):
    grid_spec = pltpu.PrefetchScalarGridSpec(
        num_scalar_prefetch=1,
        grid=(NT, NJ),
        in_specs=[
            pl.BlockSpec((TM, D), lambda i, j, t: (i, 0)),
            pl.BlockSpec((1, D, BJ), lambda i, j, t: (_clampe(t[i]), 0, j)),
            pl.BlockSpec((1, 1, BJ), lambda i, j, t: (_clampe(t[i]), 0, j)),
            pl.BlockSpec((1, BJ, L), lambda i, j, t: (_clampe(t[i]), j, 0)),
            pl.BlockSpec((1, 1, L), lambda i, j, t: (_clampe(t[i]), 0, 0)),
            pl.BlockSpec((1, 1, L), lambda i, j, t: (_clampe(t[i]), 0, 0)),
            pl.BlockSpec((1, 1, L), lambda i, j, t: (_clampe(t[i]), 0, 0)),
        ],
        out_specs=pl.BlockSpec((TM, L), lambda i, j, t: (i, 0)),
    )
    return pl.pallas_call(
        _ffn_body,
        grid_spec=grid_spec,
        out_shape=jax.ShapeDtypeStruct((PBUF, L), jnp.float32),
    )(texp, xs, W1.astype(jnp.bfloat16), b1.reshape(E, 1, I),
      W2.astype(jnp.bfloat16), b2.reshape(E, 1, L),
      ln_gamma.reshape(E, 1, L), ln_beta.reshape(E, 1, L))


def kernel(hidden_states, gate_W, W1, b1, W2, b2, ln_gamma, ln_beta):
    x = hidden_states.reshape(T, D)

    logits = pl.pallas_call(
        _router_body,
        grid=(T // 512,),
        in_specs=[
            pl.BlockSpec((512, D), lambda i: (i, 0)),
            pl.BlockSpec((D, E), lambda i: (0, 0)),
        ],
        out_specs=pl.BlockSpec((512, E), lambda i: (i, 0)),
        out_shape=jax.ShapeDtypeStruct((T, E), jnp.float32),
    )(x, gate_W.T)

    pos0, pos1, w0, w1, texp = pl.pallas_call(
        _meta_body,
        out_shape=[
            jax.ShapeDtypeStruct((T, 1), jnp.int32),
            jax.ShapeDtypeStruct((T, 1), jnp.int32),
            jax.ShapeDtypeStruct((T, 16), jnp.float32),
            jax.ShapeDtypeStruct((T, 16), jnp.float32),
            jax.ShapeDtypeStruct((NT_PAD, 1), jnp.int32),
        ],
        scratch_shapes=[pltpu.VMEM((T, E), jnp.float32)],
    )(logits)

    pos0f = pos0.reshape(T)
    pos1f = pos1.reshape(T)

    xs = _sc_scatter(x, pos0f, pos1f)

    o_sorted = _ffn_call(xs, W1, b1, W2, b2, ln_gamma, ln_beta,
                         texp.reshape(NT_PAD)[:NT])

    final = _sc_combine(o_sorted, pos0f, pos1f, w0, w1)

    return (final.reshape(hidden_states.shape[0], hidden_states.shape[1], L),
            logits)


# speedup vs baseline: 2.3166x; 1.0854x over previous
import functools

import jax
import jax.numpy as jnp
from jax import lax
from jax.experimental import pallas as pl
from jax.experimental.pallas import tpu as pltpu
from jax.experimental.pallas import tpu_sc as plsc

E = 8
KTOP = 2
D = 1024
I = 2048
L = 1024
T = 8192

TM = 512
PBUF = T * KTOP + E * TM
NT = PBUF // TM
NT_PAD = 64
BJ = 2048
NJ = I // BJ

_CHUNK = 256


def _router_body(x_ref, gwt_ref, logits_ref):
    logits_ref[...] = lax.dot_general(
        x_ref[...], gwt_ref[...], (((1,), (0,)), ((), ())),
        preferred_element_type=jnp.float32,
        precision=lax.Precision.DEFAULT)


def _meta_body(logits_ref, pos0_ref, pos1_ref, w0_ref, w1_ref, texp_ref,
               cum_ref):
    lg = logits_ref[...]
    m = jnp.max(lg, axis=1, keepdims=True)
    p = jnp.exp(lg - m)
    probs = p / jnp.sum(p, axis=1, keepdims=True)
    colid = lax.broadcasted_iota(jnp.int32, (T, E), 1)
    m1 = jnp.max(probs, axis=1, keepdims=True)
    e0 = jnp.min(jnp.where(probs == m1, colid, E), axis=1, keepdims=True)
    probs2 = jnp.where(colid == e0, -1.0, probs)
    m2 = jnp.max(probs2, axis=1, keepdims=True)
    e1 = jnp.min(jnp.where(probs2 == m2, colid, E), axis=1, keepdims=True)
    wsum = m1 + m2
    w0_ref[...] = jnp.broadcast_to(m1 / wsum, (T, 16))
    w1_ref[...] = jnp.broadcast_to(m2 / wsum, (T, 16))

    oh0 = (colid == e0).astype(jnp.float32)
    oh1 = (colid == e1).astype(jnp.float32)
    a = oh0 + oh1

    ri = lax.broadcasted_iota(jnp.int32, (_CHUNK, _CHUNK), 0)
    ci = lax.broadcasted_iota(jnp.int32, (_CHUNK, _CHUNK), 1)
    tril_ex = (ri > ci).astype(jnp.float32)

    cum_ref[...] = a

    def body(b, carry):
        ab = cum_ref[pl.ds(b * _CHUNK, _CHUNK), :]
        within = lax.dot_general(tril_ex, ab, (((1,), (0,)), ((), ())),
                                 preferred_element_type=jnp.float32)
        cum_ref[pl.ds(b * _CHUNK, _CHUNK), :] = within + carry
        return carry + jnp.sum(ab, axis=0, keepdims=True)

    counts = lax.fori_loop(0, T // _CHUNK, body, jnp.zeros((1, E), jnp.float32))
    cnt_i = counts.astype(jnp.int32)
    aligned = ((cnt_i + (TM - 1)) // TM * TM).astype(jnp.float32)
    r8 = lax.broadcasted_iota(jnp.int32, (E, E), 0)
    c8 = lax.broadcasted_iota(jnp.int32, (E, E), 1)
    lt8 = (r8 < c8).astype(jnp.float32)
    ps = lax.dot_general(aligned, lt8, (((1,), (0,)), ((), ())),
                         preferred_element_type=jnp.float32)

    cum = cum_ref[...]
    base = jnp.broadcast_to(ps, (T, E))
    pos0 = jnp.sum((cum + base) * oh0, axis=1, keepdims=True)
    pos1 = jnp.sum((cum + base) * oh1, axis=1, keepdims=True)
    pos0_ref[...] = pos0.astype(jnp.int32)
    pos1_ref[...] = pos1.astype(jnp.int32)

    ends = ps + aligned
    tb = (lax.broadcasted_iota(jnp.int32, (NT_PAD, E), 0) * TM).astype(
        jnp.float32)
    texp_ref[...] = jnp.sum((tb >= jnp.broadcast_to(ends, (NT_PAD, E)))
                            .astype(jnp.int32), axis=1, keepdims=True)


def _ffn_body(texp, xs_ref, w1_ref, b1_ref, w2_ref, b2_ref, g_ref, bt_ref,
              out_ref):
    i = pl.program_id(0)
    j = pl.program_id(1)
    e = texp[i]

    @pl.when(e < E)
    def _():
        x = xs_ref[...].astype(jnp.bfloat16)
        h = lax.dot_general(x, w1_ref[0].astype(jnp.bfloat16),
                            (((1,), (0,)), ((), ())),
                            preferred_element_type=jnp.float32)
        h = jnp.maximum(h + b1_ref[0], 0.0).astype(jnp.bfloat16)
        contrib = lax.dot_general(h, w2_ref[0].astype(jnp.bfloat16),
                                  (((1,), (0,)), ((), ())),
                                  preferred_element_type=jnp.float32)

        @pl.when(j == 0)
        def _():
            out_ref[...] = contrib

        @pl.when(j > 0)
        def _():
            out_ref[...] += contrib

        @pl.when(j == NJ - 1)
        def _():
            o = out_ref[...] + b2_ref[0]
            mu = jnp.mean(o, axis=1, keepdims=True)
            oc = o - mu
            var = jnp.mean(oc * oc, axis=1, keepdims=True)
            o = oc * lax.rsqrt(var + 1e-5)
            out_ref[...] = o * g_ref[0] + bt_ref[0]


_NC, _NS = 2, 16
_NW = _NC * _NS
_TPW = T // _NW
_SCH = 64
_CCH = 32


def _sc_scatter_body(x_hbm, pos0_hbm, pos1_hbm, xs_hbm, xv, i0v, i1v,
                     sem0, sem1):
    wid = lax.axis_index("s") * _NC + lax.axis_index("c")
    base = wid * _TPW

    def chunk(ci, _):
        tb = base + ci * _SCH
        pltpu.sync_copy(x_hbm.at[pl.ds(tb, _SCH)], xv)
        pltpu.sync_copy(pos0_hbm.at[pl.ds(tb, _SCH)], i0v)
        pltpu.sync_copy(pos1_hbm.at[pl.ds(tb, _SCH)], i1v)
        cp0 = pltpu.async_copy(xv, xs_hbm.at[i0v], sem0)
        cp1 = pltpu.async_copy(xv, xs_hbm.at[i1v], sem1)
        cp0.wait()
        cp1.wait()
        return 0

    lax.fori_loop(0, _TPW // _SCH, chunk, 0)


_sc_scatter = pl.kernel(
    _sc_scatter_body,
    out_type=jax.ShapeDtypeStruct((PBUF, D), jnp.float32),
    mesh=plsc.VectorSubcoreMesh(core_axis_name="c", subcore_axis_name="s"),
    scratch_types=[
        pltpu.VMEM((_SCH, D), jnp.float32),
        pltpu.VMEM((_SCH,), jnp.int32),
        pltpu.VMEM((_SCH,), jnp.int32),
        pltpu.SemaphoreType.DMA,
        pltpu.SemaphoreType.DMA,
    ],
)


def _sc_combine_body(o_hbm, pos0_hbm, pos1_hbm, w0_hbm, w1_hbm, out_hbm,
                     i0v, i1v, w0v, w1v, b0, b1, ov, sem0, sem1):
    wid = lax.axis_index("s") * _NC + lax.axis_index("c")
    base = wid * _TPW

    def chunk(ci, _):
        tb = base + ci * _CCH
        pltpu.sync_copy(pos0_hbm.at[pl.ds(tb, _CCH)], i0v)
        pltpu.sync_copy(pos1_hbm.at[pl.ds(tb, _CCH)], i1v)
        pltpu.sync_copy(w0_hbm.at[pl.ds(tb, _CCH)], w0v)
        pltpu.sync_copy(w1_hbm.at[pl.ds(tb, _CCH)], w1v)
        cp0 = pltpu.async_copy(o_hbm.at[i0v], b0, sem0)
        cp1 = pltpu.async_copy(o_hbm.at[i1v], b1, sem1)
        cp0.wait()
        cp1.wait()

        def row(r, _):
            w0r = w0v[r, pl.ds(0, 16)]
            w1r = w1v[r, pl.ds(0, 16)]
            for c in range(L // 16):
                ov[r, pl.ds(c * 16, 16)] = (
                    w0r * b0[r, pl.ds(c * 16, 16)]
                    + w1r * b1[r, pl.ds(c * 16, 16)])
            return 0

        lax.fori_loop(0, _CCH, row, 0)
        pltpu.sync_copy(ov, out_hbm.at[pl.ds(tb, _CCH)])
        return 0

    lax.fori_loop(0, _TPW // _CCH, chunk, 0)


_sc_combine = pl.kernel(
    _sc_combine_body,
    out_type=jax.ShapeDtypeStruct((T, L), jnp.float32),
    mesh=plsc.VectorSubcoreMesh(core_axis_name="c", subcore_axis_name="s"),
    scratch_types=[
        pltpu.VMEM((_CCH,), jnp.int32),
        pltpu.VMEM((_CCH,), jnp.int32),
        pltpu.VMEM((_CCH, 16), jnp.float32),
        pltpu.VMEM((_CCH, 16), jnp.float32),
        pltpu.VMEM((_CCH, L), jnp.float32),
        pltpu.VMEM((_CCH, L), jnp.float32),
        pltpu.VMEM((_CCH, L), jnp.float32),
        pltpu.SemaphoreType.DMA,
        pltpu.SemaphoreType.DMA,
    ],
)


def _clampe(e):
    return jnp.minimum(e, E - 1)


def _ffn_call(xs, W1, b1, W2, b2, ln_gamma, ln_beta, texp):
    grid_spec = pltpu.PrefetchScalarGridSpec(
        num_scalar_prefetch=1,
        grid=(NT, NJ),
        in_specs=[
            pl.BlockSpec((TM, D), lambda i, j, t: (i, 0)),
            pl.BlockSpec((1, D, BJ), lambda i, j, t: (_clampe(t[i]), 0, j)),
            pl.BlockSpec((1, 1, BJ), lambda i, j, t: (_clampe(t[i]), 0, j)),
            pl.BlockSpec((1, BJ, L), lambda i, j, t: (_clampe(t[i]), j, 0)),
            pl.BlockSpec((1, 1, L), lambda i, j, t: (_clampe(t[i]), 0, 0)),
            pl.BlockSpec((1, 1, L), lambda i, j, t: (_clampe(t[i]), 0, 0)),
            pl.BlockSpec((1, 1, L), lambda i, j, t: (_clampe(t[i]), 0, 0)),
        ],
        out_specs=pl.BlockSpec((TM, L), lambda i, j, t: (i, 0)),
    )
    return pl.pallas_call(
        _ffn_body,
        grid_spec=grid_spec,
        out_shape=jax.ShapeDtypeStruct((PBUF, L), jnp.float32),
    )(texp, xs, W1, b1.reshape(E, 1, I), W2, b2.reshape(E, 1, L),
      ln_gamma.reshape(E, 1, L), ln_beta.reshape(E, 1, L))


def kernel(hidden_states, gate_W, W1, b1, W2, b2, ln_gamma, ln_beta):
    x = hidden_states.reshape(T, D)

    logits = pl.pallas_call(
        _router_body,
        grid=(T // 512,),
        in_specs=[
            pl.BlockSpec((512, D), lambda i: (i, 0)),
            pl.BlockSpec((D, E), lambda i: (0, 0)),
        ],
        out_specs=pl.BlockSpec((512, E), lambda i: (i, 0)),
        out_shape=jax.ShapeDtypeStruct((T, E), jnp.float32),
    )(x, gate_W.T)

    pos0, pos1, w0, w1, texp = pl.pallas_call(
        _meta_body,
        out_shape=[
            jax.ShapeDtypeStruct((T, 1), jnp.int32),
            jax.ShapeDtypeStruct((T, 1), jnp.int32),
            jax.ShapeDtypeStruct((T, 16), jnp.float32),
            jax.ShapeDtypeStruct((T, 16), jnp.float32),
            jax.ShapeDtypeStruct((NT_PAD, 1), jnp.int32),
        ],
        scratch_shapes=[pltpu.VMEM((T, E), jnp.float32)],
    )(logits)

    pos0f = pos0.reshape(T)
    pos1f = pos1.reshape(T)

    xs = _sc_scatter(x, pos0f, pos1f)

    o_sorted = _ffn_call(xs, W1, b1, W2, b2, ln_gamma, ln_beta,
                         texp.reshape(NT_PAD)[:NT])

    final = _sc_combine(o_sorted, pos0f, pos1f, w0, w1)

    return (final.reshape(hidden_states.shape[0], hidden_states.shape[1], L),
            logits)


# speedup vs baseline: 2.3207x; 1.0017x over previous
import functools

import jax
import jax.numpy as jnp
from jax import lax
from jax.experimental import pallas as pl
from jax.experimental.pallas import tpu as pltpu
from jax.experimental.pallas import tpu_sc as plsc

E = 8
KTOP = 2
D = 1024
I = 2048
L = 1024
T = 8192

TM = 512
PBUF = T * KTOP + E * TM
NT = PBUF // TM
NT_PAD = 64
BJ = 2048
NJ = I // BJ

_CHUNK = 256


def _router_body(x_ref, gwt_ref, logits_ref):
    logits_ref[...] = lax.dot_general(
        x_ref[...], gwt_ref[...], (((1,), (0,)), ((), ())),
        preferred_element_type=jnp.float32,
        precision=lax.Precision.DEFAULT)


def _meta_body(logits_ref, pos0_ref, pos1_ref, w0_ref, w1_ref, texp_ref,
               cum_ref):
    lg = logits_ref[...]
    m = jnp.max(lg, axis=1, keepdims=True)
    p = jnp.exp(lg - m)
    probs = p / jnp.sum(p, axis=1, keepdims=True)
    colid = lax.broadcasted_iota(jnp.int32, (T, E), 1)
    m1 = jnp.max(probs, axis=1, keepdims=True)
    e0 = jnp.min(jnp.where(probs == m1, colid, E), axis=1, keepdims=True)
    probs2 = jnp.where(colid == e0, -1.0, probs)
    m2 = jnp.max(probs2, axis=1, keepdims=True)
    e1 = jnp.min(jnp.where(probs2 == m2, colid, E), axis=1, keepdims=True)
    wsum = m1 + m2
    w0_ref[...] = jnp.broadcast_to(m1 / wsum, (T, 16))
    w1_ref[...] = jnp.broadcast_to(m2 / wsum, (T, 16))

    oh0 = (colid == e0).astype(jnp.float32)
    oh1 = (colid == e1).astype(jnp.float32)
    a = oh0 + oh1

    ri = lax.broadcasted_iota(jnp.int32, (_CHUNK, _CHUNK), 0)
    ci = lax.broadcasted_iota(jnp.int32, (_CHUNK, _CHUNK), 1)
    tril_ex = (ri > ci).astype(jnp.float32)

    cum_ref[...] = a

    def body(b, carry):
        ab = cum_ref[pl.ds(b * _CHUNK, _CHUNK), :]
        within = lax.dot_general(tril_ex, ab, (((1,), (0,)), ((), ())),
                                 preferred_element_type=jnp.float32)
        cum_ref[pl.ds(b * _CHUNK, _CHUNK), :] = within + carry
        return carry + jnp.sum(ab, axis=0, keepdims=True)

    counts = lax.fori_loop(0, T // _CHUNK, body, jnp.zeros((1, E), jnp.float32))
    cnt_i = counts.astype(jnp.int32)
    aligned = ((cnt_i + (TM - 1)) // TM * TM).astype(jnp.float32)
    r8 = lax.broadcasted_iota(jnp.int32, (E, E), 0)
    c8 = lax.broadcasted_iota(jnp.int32, (E, E), 1)
    lt8 = (r8 < c8).astype(jnp.float32)
    ps = lax.dot_general(aligned, lt8, (((1,), (0,)), ((), ())),
                         preferred_element_type=jnp.float32)

    cum = cum_ref[...]
    base = jnp.broadcast_to(ps, (T, E))
    pos0 = jnp.sum((cum + base) * oh0, axis=1, keepdims=True)
    pos1 = jnp.sum((cum + base) * oh1, axis=1, keepdims=True)
    pos0_ref[...] = pos0.astype(jnp.int32)
    pos1_ref[...] = pos1.astype(jnp.int32)

    ends = ps + aligned
    tb = (lax.broadcasted_iota(jnp.int32, (NT_PAD, E), 0) * TM).astype(
        jnp.float32)
    texp_ref[...] = jnp.sum((tb >= jnp.broadcast_to(ends, (NT_PAD, E)))
                            .astype(jnp.int32), axis=1, keepdims=True)


def _ffn_body(texp, xs_ref, w1_ref, b1_ref, w2_ref, b2_ref, g_ref, bt_ref,
              out_ref):
    i = pl.program_id(0)
    j = pl.program_id(1)
    e = texp[i]

    @pl.when(e < E)
    def _():
        x = xs_ref[...].astype(jnp.bfloat16)
        h = lax.dot_general(x, w1_ref[0].astype(jnp.bfloat16),
                            (((1,), (0,)), ((), ())),
                            preferred_element_type=jnp.float32)
        h = jnp.maximum(h + b1_ref[0], 0.0).astype(jnp.bfloat16)
        contrib = lax.dot_general(h, w2_ref[0].astype(jnp.bfloat16),
                                  (((1,), (0,)), ((), ())),
                                  preferred_element_type=jnp.float32)

        @pl.when(j == 0)
        def _():
            out_ref[...] = contrib

        @pl.when(j > 0)
        def _():
            out_ref[...] += contrib

        @pl.when(j == NJ - 1)
        def _():
            o = out_ref[...] + b2_ref[0]
            mu = jnp.mean(o, axis=1, keepdims=True)
            oc = o - mu
            var = jnp.mean(oc * oc, axis=1, keepdims=True)
            o = oc * lax.rsqrt(var + 1e-5)
            out_ref[...] = o * g_ref[0] + bt_ref[0]


_NC, _NS = 2, 16
_NW = _NC * _NS
_TPW = T // _NW
_SCH = 32
_SNC = _TPW // _SCH
_CCH = 16
_CNC = _TPW // _CCH


def _sc_scatter_body(x_hbm, pos0_hbm, pos1_hbm, xs_hbm, xv,
                     i0a, i0b, i1a, i1b, semx0, semx1, sems0, sems1):
    wid = lax.axis_index("s") * _NC + lax.axis_index("c")
    base = wid * _TPW
    i0 = [i0a, i0b]
    i1 = [i1a, i1b]
    semx = [semx0, semx1]
    sems = [sems0, sems1]
    scat = [None, None]

    def load(ci, buf):
        if scat[buf] is not None:
            scat[buf][0].wait()
            scat[buf][1].wait()
            scat[buf] = None
        tb = base + ci * _SCH
        pltpu.sync_copy(pos0_hbm.at[pl.ds(tb, _SCH)], i0[buf])
        pltpu.sync_copy(pos1_hbm.at[pl.ds(tb, _SCH)], i1[buf])
        return pltpu.async_copy(x_hbm.at[pl.ds(tb, _SCH)], xv.at[buf],
                                semx[buf])

    ld = load(0, 0)
    for ci in range(_SNC):
        buf = ci % 2
        nld = load(ci + 1, 1 - buf) if ci + 1 < _SNC else None
        ld.wait()
        cp0 = pltpu.async_copy(xv.at[buf], xs_hbm.at[i0[buf]], sems[buf])
        cp1 = pltpu.async_copy(xv.at[buf], xs_hbm.at[i1[buf]], sems[buf])
        scat[buf] = (cp0, cp1)
        ld = nld
    for pair in scat:
        if pair is not None:
            pair[0].wait()
            pair[1].wait()


_sc_scatter = pl.kernel(
    _sc_scatter_body,
    out_type=jax.ShapeDtypeStruct((PBUF, D), jnp.float32),
    mesh=plsc.VectorSubcoreMesh(core_axis_name="c", subcore_axis_name="s"),
    scratch_types=[
        pltpu.VMEM((2, _SCH, D), jnp.float32),
        pltpu.VMEM((_SCH,), jnp.int32),
        pltpu.VMEM((_SCH,), jnp.int32),
        pltpu.VMEM((_SCH,), jnp.int32),
        pltpu.VMEM((_SCH,), jnp.int32),
        pltpu.SemaphoreType.DMA,
        pltpu.SemaphoreType.DMA,
        pltpu.SemaphoreType.DMA,
        pltpu.SemaphoreType.DMA,
    ],
)


def _sc_combine_body(o_hbm, pos0_hbm, pos1_hbm, w0_hbm, w1_hbm, out_hbm,
                     i0a, i0b, i1a, i1b, w0v, w1v, b0, b1, ov,
                     semg0, semg1, semw0, semw1):
    wid = lax.axis_index("s") * _NC + lax.axis_index("c")
    base = wid * _TPW
    i0 = [i0a, i0b]
    i1 = [i1a, i1b]
    semg = [semg0, semg1]
    semw = [semw0, semw1]

    def fetch(ci, buf):
        tb = base + ci * _CCH
        pltpu.sync_copy(pos0_hbm.at[pl.ds(tb, _CCH)], i0[buf])
        pltpu.sync_copy(pos1_hbm.at[pl.ds(tb, _CCH)], i1[buf])
        pltpu.sync_copy(w0_hbm.at[pl.ds(tb, _CCH)], w0v.at[buf])
        pltpu.sync_copy(w1_hbm.at[pl.ds(tb, _CCH)], w1v.at[buf])
        g0 = pltpu.async_copy(o_hbm.at[i0[buf]], b0.at[buf], semg[buf])
        g1 = pltpu.async_copy(o_hbm.at[i1[buf]], b1.at[buf], semg[buf])
        return (g0, g1)

    wr = [None, None]
    g = fetch(0, 0)
    for ci in range(_CNC):
        buf = ci % 2
        ng = fetch(ci + 1, 1 - buf) if ci + 1 < _CNC else None
        g[0].wait()
        g[1].wait()
        if wr[buf] is not None:
            wr[buf].wait()

        def row(r, _):
            w0r = w0v[buf, r, pl.ds(0, 16)]
            w1r = w1v[buf, r, pl.ds(0, 16)]
            for c in range(L // 16):
                ov[buf, r, pl.ds(c * 16, 16)] = (
                    w0r * b0[buf, r, pl.ds(c * 16, 16)]
                    + w1r * b1[buf, r, pl.ds(c * 16, 16)])
            return 0

        lax.fori_loop(0, _CCH, row, 0)
        tb = base + ci * _CCH
        wr[buf] = pltpu.async_copy(ov.at[buf], out_hbm.at[pl.ds(tb, _CCH)],
                                   semw[buf])
        g = ng
    for w in wr:
        if w is not None:
            w.wait()


_sc_combine = pl.kernel(
    _sc_combine_body,
    out_type=jax.ShapeDtypeStruct((T, L), jnp.float32),
    mesh=plsc.VectorSubcoreMesh(core_axis_name="c", subcore_axis_name="s"),
    scratch_types=[
        pltpu.VMEM((_CCH,), jnp.int32),
        pltpu.VMEM((_CCH,), jnp.int32),
        pltpu.VMEM((_CCH,), jnp.int32),
        pltpu.VMEM((_CCH,), jnp.int32),
        pltpu.VMEM((2, _CCH, 16), jnp.float32),
        pltpu.VMEM((2, _CCH, 16), jnp.float32),
        pltpu.VMEM((2, _CCH, L), jnp.float32),
        pltpu.VMEM((2, _CCH, L), jnp.float32),
        pltpu.VMEM((2, _CCH, L), jnp.float32),
        pltpu.SemaphoreType.DMA,
        pltpu.SemaphoreType.DMA,
        pltpu.SemaphoreType.DMA,
        pltpu.SemaphoreType.DMA,
    ],
)


def _clampe(e):
    return jnp.minimum(e, E - 1)


def _ffn_call(xs, W1, b1, W2, b2, ln_gamma, ln_beta, texp):
    grid_spec = pltpu.PrefetchScalarGridSpec(
        num_scalar_prefetch=1,
        grid=(NT, NJ),
        in_specs=[
            pl.BlockSpec((TM, D), lambda i, j, t: (i, 0)),
            pl.BlockSpec((1, D, BJ), lambda i, j, t: (_clampe(t[i]), 0, j)),
            pl.BlockSpec((1, 1, BJ), lambda i, j, t: (_clampe(t[i]), 0, j)),
            pl.BlockSpec((1, BJ, L), lambda i, j, t: (_clampe(t[i]), j, 0)),
            pl.BlockSpec((1, 1, L), lambda i, j, t: (_clampe(t[i]), 0, 0)),
            pl.BlockSpec((1, 1, L), lambda i, j, t: (_clampe(t[i]), 0, 0)),
            pl.BlockSpec((1, 1, L), lambda i, j, t: (_clampe(t[i]), 0, 0)),
        ],
        out_specs=pl.BlockSpec((TM, L), lambda i, j, t: (i, 0)),
    )
    return pl.pallas_call(
        _ffn_body,
        grid_spec=grid_spec,
        out_shape=jax.ShapeDtypeStruct((PBUF, L), jnp.float32),
    )(texp, xs, W1, b1.reshape(E, 1, I), W2, b2.reshape(E, 1, L),
      ln_gamma.reshape(E, 1, L), ln_beta.reshape(E, 1, L))


def kernel(hidden_states, gate_W, W1, b1, W2, b2, ln_gamma, ln_beta):
    x = hidden_states.reshape(T, D)

    logits = pl.pallas_call(
        _router_body,
        grid=(T // 512,),
        in_specs=[
            pl.BlockSpec((512, D), lambda i: (i, 0)),
            pl.BlockSpec((D, E), lambda i: (0, 0)),
        ],
        out_specs=pl.BlockSpec((512, E), lambda i: (i, 0)),
        out_shape=jax.ShapeDtypeStruct((T, E), jnp.float32),
    )(x, gate_W.T)

    pos0, pos1, w0, w1, texp = pl.pallas_call(
        _meta_body,
        out_shape=[
            jax.ShapeDtypeStruct((T, 1), jnp.int32),
            jax.ShapeDtypeStruct((T, 1), jnp.int32),
            jax.ShapeDtypeStruct((T, 16), jnp.float32),
            jax.ShapeDtypeStruct((T, 16), jnp.float32),
            jax.ShapeDtypeStruct((NT_PAD, 1), jnp.int32),
        ],
        scratch_shapes=[pltpu.VMEM((T, E), jnp.float32)],
    )(logits)

    pos0f = pos0.reshape(T)
    pos1f = pos1.reshape(T)

    xs = _sc_scatter(x, pos0f, pos1f)

    o_sorted = _ffn_call(xs, W1, b1, W2, b2, ln_gamma, ln_beta,
                         texp.reshape(NT_PAD)[:NT])

    final = _sc_combine(o_sorted, pos0f, pos1f, w0, w1)

    return (final.reshape(hidden_states.shape[0], hidden_states.shape[1], L),
            logits)


# speedup vs baseline: 2.4791x; 1.0683x over previous
import functools

import jax
import jax.numpy as jnp
from jax import lax
from jax.experimental import pallas as pl
from jax.experimental.pallas import tpu as pltpu
from jax.experimental.pallas import tpu_sc as plsc

E = 8
KTOP = 2
D = 1024
I = 2048
L = 1024
T = 8192

TM = 512
PBUF = T * KTOP + E * TM
NT = PBUF // TM
NT_PAD = 64
BJ = 2048
NJ = I // BJ

_CHUNK = 256


def _router_body(x_ref, gwt_ref, logits_ref):
    logits_ref[...] = lax.dot_general(
        x_ref[...], gwt_ref[...], (((1,), (0,)), ((), ())),
        preferred_element_type=jnp.float32,
        precision=lax.Precision.DEFAULT)


def _meta_body(logits_ref, pos0_ref, pos1_ref, w0_ref, w1_ref, texp_ref,
               cum_ref):
    lg = logits_ref[...]
    m = jnp.max(lg, axis=1, keepdims=True)
    p = jnp.exp(lg - m)
    probs = p / jnp.sum(p, axis=1, keepdims=True)
    colid = lax.broadcasted_iota(jnp.int32, (T, E), 1)
    m1 = jnp.max(probs, axis=1, keepdims=True)
    e0 = jnp.min(jnp.where(probs == m1, colid, E), axis=1, keepdims=True)
    probs2 = jnp.where(colid == e0, -1.0, probs)
    m2 = jnp.max(probs2, axis=1, keepdims=True)
    e1 = jnp.min(jnp.where(probs2 == m2, colid, E), axis=1, keepdims=True)
    wsum = m1 + m2
    w0_ref[...] = jnp.broadcast_to(m1 / wsum, (T, 16))
    w1_ref[...] = jnp.broadcast_to(m2 / wsum, (T, 16))

    oh0 = (colid == e0).astype(jnp.float32)
    oh1 = (colid == e1).astype(jnp.float32)
    a = oh0 + oh1

    ri = lax.broadcasted_iota(jnp.int32, (_CHUNK, _CHUNK), 0)
    ci = lax.broadcasted_iota(jnp.int32, (_CHUNK, _CHUNK), 1)
    tril_ex = (ri > ci).astype(jnp.float32)

    cum_ref[...] = a

    def body(b, carry):
        ab = cum_ref[pl.ds(b * _CHUNK, _CHUNK), :]
        within = lax.dot_general(tril_ex, ab, (((1,), (0,)), ((), ())),
                                 preferred_element_type=jnp.float32)
        cum_ref[pl.ds(b * _CHUNK, _CHUNK), :] = within + carry
        return carry + jnp.sum(ab, axis=0, keepdims=True)

    counts = lax.fori_loop(0, T // _CHUNK, body, jnp.zeros((1, E), jnp.float32))
    cnt_i = counts.astype(jnp.int32)
    aligned = ((cnt_i + (TM - 1)) // TM * TM).astype(jnp.float32)
    r8 = lax.broadcasted_iota(jnp.int32, (E, E), 0)
    c8 = lax.broadcasted_iota(jnp.int32, (E, E), 1)
    lt8 = (r8 < c8).astype(jnp.float32)
    ps = lax.dot_general(aligned, lt8, (((1,), (0,)), ((), ())),
                         preferred_element_type=jnp.float32)

    cum = cum_ref[...]
    base = jnp.broadcast_to(ps, (T, E))
    pos0 = jnp.sum((cum + base) * oh0, axis=1, keepdims=True)
    pos1 = jnp.sum((cum + base) * oh1, axis=1, keepdims=True)
    pos0_ref[...] = pos0.astype(jnp.int32)
    pos1_ref[...] = pos1.astype(jnp.int32)

    ends = ps + aligned
    tb = (lax.broadcasted_iota(jnp.int32, (NT_PAD, E), 0) * TM).astype(
        jnp.float32)
    texp_ref[...] = jnp.sum((tb >= jnp.broadcast_to(ends, (NT_PAD, E)))
                            .astype(jnp.int32), axis=1, keepdims=True)


def _ffn_body(texp, xs_ref, w1_ref, b1_ref, w2_ref, b2_ref, g_ref, bt_ref,
              out_ref):
    i = pl.program_id(0)
    j = pl.program_id(1)
    e = texp[i]

    @pl.when(e < E)
    def _():
        x = xs_ref[...].astype(jnp.bfloat16)
        h = lax.dot_general(x, w1_ref[0].astype(jnp.bfloat16),
                            (((1,), (0,)), ((), ())),
                            preferred_element_type=jnp.float32)
        h = jnp.maximum(h + b1_ref[0], 0.0).astype(jnp.bfloat16)
        contrib = lax.dot_general(h, w2_ref[0].astype(jnp.bfloat16),
                                  (((1,), (0,)), ((), ())),
                                  preferred_element_type=jnp.float32)

        @pl.when(j == 0)
        def _():
            out_ref[...] = contrib

        @pl.when(j > 0)
        def _():
            out_ref[...] += contrib

        @pl.when(j == NJ - 1)
        def _():
            o = out_ref[...] + b2_ref[0]
            mu = jnp.mean(o, axis=1, keepdims=True)
            oc = o - mu
            var = jnp.mean(oc * oc, axis=1, keepdims=True)
            o = oc * lax.rsqrt(var + 1e-5)
            out_ref[...] = o * g_ref[0] + bt_ref[0]


_NC, _NS = 2, 16
_NW = _NC * _NS
_TPW = T // _NW
_SCH = 32
_SNC = _TPW // _SCH
_CCH = 16
_CNC = _TPW // _CCH


def _sc_scatter_body(x_hbm, pos0_hbm, pos1_hbm, xs_hbm, xv, i0all, i1all,
                     semx0, semx1, sems0, sems1):
    wid = lax.axis_index("s") * _NC + lax.axis_index("c")
    base = wid * _TPW
    pltpu.sync_copy(pos0_hbm.at[wid], i0all)
    pltpu.sync_copy(pos1_hbm.at[wid], i1all)
    semx = [semx0, semx1]
    sems = [sems0, sems1]
    scat = [None, None]

    def load(ci, buf):
        if scat[buf] is not None:
            scat[buf][0].wait()
            scat[buf][1].wait()
            scat[buf] = None
        return pltpu.async_copy(x_hbm.at[pl.ds(base + ci * _SCH, _SCH)],
                                xv.at[buf], semx[buf])

    ld = load(0, 0)
    for ci in range(_SNC):
        buf = ci % 2
        nld = load(ci + 1, 1 - buf) if ci + 1 < _SNC else None
        ld.wait()
        cp0 = pltpu.async_copy(xv.at[buf], xs_hbm.at[i0all.at[ci]],
                               sems[buf])
        cp1 = pltpu.async_copy(xv.at[buf], xs_hbm.at[i1all.at[ci]],
                               sems[buf])
        scat[buf] = (cp0, cp1)
        ld = nld
    for pair in scat:
        if pair is not None:
            pair[0].wait()
            pair[1].wait()


_sc_scatter = pl.kernel(
    _sc_scatter_body,
    out_type=jax.ShapeDtypeStruct((PBUF, D), jnp.float32),
    mesh=plsc.VectorSubcoreMesh(core_axis_name="c", subcore_axis_name="s"),
    scratch_types=[
        pltpu.VMEM((2, _SCH, D), jnp.float32),
        pltpu.VMEM((_SNC, _SCH), jnp.int32),
        pltpu.VMEM((_SNC, _SCH), jnp.int32),
        pltpu.SemaphoreType.DMA,
        pltpu.SemaphoreType.DMA,
        pltpu.SemaphoreType.DMA,
        pltpu.SemaphoreType.DMA,
    ],
)


def _sc_combine_body(o_hbm, pos0_hbm, pos1_hbm, w0_hbm, w1_hbm, out_hbm,
                     i0all, i1all, w0all, w1all, b0, b1, ov,
                     semg0, semg1, semw0, semw1):
    wid = lax.axis_index("s") * _NC + lax.axis_index("c")
    base = wid * _TPW
    pltpu.sync_copy(pos0_hbm.at[wid], i0all)
    pltpu.sync_copy(pos1_hbm.at[wid], i1all)
    pltpu.sync_copy(w0_hbm.at[wid], w0all)
    pltpu.sync_copy(w1_hbm.at[wid], w1all)
    semg = [semg0, semg1]
    semw = [semw0, semw1]

    def fetch(ci, buf):
        g0 = pltpu.async_copy(o_hbm.at[i0all.at[ci]], b0.at[buf], semg[buf])
        g1 = pltpu.async_copy(o_hbm.at[i1all.at[ci]], b1.at[buf], semg[buf])
        return (g0, g1)

    wr = [None]
    g = fetch(0, 0)
    for ci in range(_CNC):
        buf = ci % 2
        ng = fetch(ci + 1, 1 - buf) if ci + 1 < _CNC else None
        g[0].wait()
        g[1].wait()
        if wr[0] is not None:
            wr[0].wait()

        def row(r, _):
            rw = (ci * _CCH + r) * 16
            w0r = w0all[pl.ds(rw, 16)]
            w1r = w1all[pl.ds(rw, 16)]
            for c in range(L // 16):
                ov[r, pl.ds(c * 16, 16)] = (
                    w0r * b0[buf, r, pl.ds(c * 16, 16)]
                    + w1r * b1[buf, r, pl.ds(c * 16, 16)])
            return 0

        lax.fori_loop(0, _CCH, row, 0)
        tb = base + ci * _CCH
        wr[0] = pltpu.async_copy(ov, out_hbm.at[pl.ds(tb, _CCH)], semw[0])
        g = ng
    if wr[0] is not None:
        wr[0].wait()


_sc_combine = pl.kernel(
    _sc_combine_body,
    out_type=jax.ShapeDtypeStruct((T, L), jnp.float32),
    mesh=plsc.VectorSubcoreMesh(core_axis_name="c", subcore_axis_name="s"),
    scratch_types=[
        pltpu.VMEM((_CNC, _CCH), jnp.int32),
        pltpu.VMEM((_CNC, _CCH), jnp.int32),
        pltpu.VMEM((_TPW * 16,), jnp.float32),
        pltpu.VMEM((_TPW * 16,), jnp.float32),
        pltpu.VMEM((2, _CCH, L), jnp.float32),
        pltpu.VMEM((2, _CCH, L), jnp.float32),
        pltpu.VMEM((_CCH, L), jnp.float32),
        pltpu.SemaphoreType.DMA,
        pltpu.SemaphoreType.DMA,
        pltpu.SemaphoreType.DMA,
        pltpu.SemaphoreType.DMA,
    ],
)


def _clampe(e):
    return jnp.minimum(e, E - 1)


def _ffn_call(xs, W1, b1, W2, b2, ln_gamma, ln_beta, texp):
    grid_spec = pltpu.PrefetchScalarGridSpec(
        num_scalar_prefetch=1,
        grid=(NT, NJ),
        in_specs=[
            pl.BlockSpec((TM, D), lambda i, j, t: (i, 0)),
            pl.BlockSpec((1, D, BJ), lambda i, j, t: (_clampe(t[i]), 0, j)),
            pl.BlockSpec((1, 1, BJ), lambda i, j, t: (_clampe(t[i]), 0, j)),
            pl.BlockSpec((1, BJ, L), lambda i, j, t: (_clampe(t[i]), j, 0)),
            pl.BlockSpec((1, 1, L), lambda i, j, t: (_clampe(t[i]), 0, 0)),
            pl.BlockSpec((1, 1, L), lambda i, j, t: (_clampe(t[i]), 0, 0)),
            pl.BlockSpec((1, 1, L), lambda i, j, t: (_clampe(t[i]), 0, 0)),
        ],
        out_specs=pl.BlockSpec((TM, L), lambda i, j, t: (i, 0)),
    )
    return pl.pallas_call(
        _ffn_body,
        grid_spec=grid_spec,
        out_shape=jax.ShapeDtypeStruct((PBUF, L), jnp.float32),
    )(texp, xs, W1, b1.reshape(E, 1, I), W2, b2.reshape(E, 1, L),
      ln_gamma.reshape(E, 1, L), ln_beta.reshape(E, 1, L))


def kernel(hidden_states, gate_W, W1, b1, W2, b2, ln_gamma, ln_beta):
    x = hidden_states.reshape(T, D)

    logits = pl.pallas_call(
        _router_body,
        grid=(T // 512,),
        in_specs=[
            pl.BlockSpec((512, D), lambda i: (i, 0)),
            pl.BlockSpec((D, E), lambda i: (0, 0)),
        ],
        out_specs=pl.BlockSpec((512, E), lambda i: (i, 0)),
        out_shape=jax.ShapeDtypeStruct((T, E), jnp.float32),
    )(x, gate_W.T)

    pos0, pos1, w0, w1, texp = pl.pallas_call(
        _meta_body,
        out_shape=[
            jax.ShapeDtypeStruct((T, 1), jnp.int32),
            jax.ShapeDtypeStruct((T, 1), jnp.int32),
            jax.ShapeDtypeStruct((T, 16), jnp.float32),
            jax.ShapeDtypeStruct((T, 16), jnp.float32),
            jax.ShapeDtypeStruct((NT_PAD, 1), jnp.int32),
        ],
        scratch_shapes=[pltpu.VMEM((T, E), jnp.float32)],
    )(logits)

    pos0f = pos0.reshape(T)
    pos1f = pos1.reshape(T)

    xs = _sc_scatter(x, pos0f.reshape(_NW, _SNC, _SCH),
                     pos1f.reshape(_NW, _SNC, _SCH))

    o_sorted = _ffn_call(xs, W1, b1, W2, b2, ln_gamma, ln_beta,
                         texp.reshape(NT_PAD)[:NT])

    final = _sc_combine(o_sorted, pos0f.reshape(_NW, _CNC, _CCH),
                        pos1f.reshape(_NW, _CNC, _CCH),
                        w0.reshape(_NW, _TPW * 16), w1.reshape(_NW, _TPW * 16))

    return (final.reshape(hidden_states.shape[0], hidden_states.shape[1], L),
            logits)
